# baseline (speedup 1.0000x reference)
import functools

import jax
import jax.numpy as jnp
from jax import lax
from jax.experimental import pallas as pl
from jax.experimental.pallas import tpu as pltpu
from jax.experimental.pallas import tpu_sc as plsc

_N = 10000
_E = 320000
_D = 128
_T = 4

_NC = 2
_NS = 16
_CH = 128
_NCHUNK = 80
_NPH = 2
_HC = _NCHUNK // _NPH
_EPT = _NCHUNK * _CH
_EPAD = _NC * _NS * _EPT
_NP = 10240
_RPT = _NP // _NS

_R = 1000


def _agg_one_table(do_gather, x_hbm, src_hbm, dst_hbm, zeros_hbm, out_hbm,
                   sidx, didx, rows0, rows1, sem0, sem1, acc, wid, c, r0):
    pltpu.sync_copy(zeros_hbm.at[pl.ds(r0, _RPT)], acc.at[pl.ds(r0, _RPT)])
    plsc.subcore_barrier()

    for p in range(_NPH):
        if do_gather:
            pltpu.sync_copy(src_hbm.at[wid, pl.ds(p * _HC, _HC)], sidx)
        pltpu.sync_copy(dst_hbm.at[wid, pl.ds(p * _HC, _HC)], didx)

        if do_gather:
            def fire(j, rows, sem):
                pltpu.async_copy(x_hbm.at[sidx.at[j]], rows, sem)

            def drain(rows, sem):
                pltpu.make_async_copy(x_hbm.at[sidx.at[0]], rows, sem).wait()

            fire(0, rows0, sem0)
            fire(1, rows1, sem1)

            def group(gi, carry):
                drain(rows0, sem0)
                pltpu.sync_copy(rows0, acc.at[didx.at[2 * gi]], add=True)
                fire(2 * gi + 2, rows0, sem0)
                drain(rows1, sem1)
                pltpu.sync_copy(rows1, acc.at[didx.at[2 * gi + 1]], add=True)
                fire(2 * gi + 3, rows1, sem1)
                return carry

            lax.fori_loop(0, _HC // 2 - 1, group, 0)
            drain(rows0, sem0)
            pltpu.sync_copy(rows0, acc.at[didx.at[_HC - 2]], add=True)
            drain(rows1, sem1)
            pltpu.sync_copy(rows1, acc.at[didx.at[_HC - 1]], add=True)
        else:
            def chunk(j, carry):
                pltpu.sync_copy(rows0, acc.at[didx.at[j]], add=True)
                return carry

            lax.fori_loop(0, _HC, chunk, 0)

    plsc.subcore_barrier()
    pltpu.sync_copy(acc.at[pl.ds(r0, _RPT)],
                    out_hbm.at[pl.ds(c * _NP + r0, _RPT)])


_SC_SCRATCH = [
    pltpu.VMEM((_HC, _CH), jnp.int32),
    pltpu.VMEM((_HC, _CH), jnp.int32),
    pltpu.VMEM((_CH, _D), jnp.float32),
    pltpu.VMEM((_CH, _D), jnp.float32),
    pltpu.SemaphoreType.DMA,
    pltpu.SemaphoreType.DMA,
    pltpu.VMEM_SHARED((_NP, _D), jnp.float32),
]


def _sc_agg_body(do_gather, x_hbm, src_hbm, dst_hbm, zeros_hbm, out_hbm,
                 sidx, didx, rows0, rows1, sem0, sem1, acc):
    c = lax.axis_index("c")
    s = lax.axis_index("s")
    if not do_gather:
        pltpu.sync_copy(x_hbm, rows0)
    _agg_one_table(do_gather, x_hbm, src_hbm, dst_hbm, zeros_hbm, out_hbm,
                   sidx, didx, rows0, rows1, sem0, sem1, acc,
                   c * _NS + s, c, s * _RPT)


def _sc_aggx4_body(x0, x1, x2, x3, src_hbm, dst_hbm, zeros_hbm,
                   o0, o1, o2, o3, sidx, didx, rows0, rows1, sem0, sem1, acc):
    c = lax.axis_index("c")
    s = lax.axis_index("s")
    for x_hbm, out_hbm in ((x0, o0), (x1, o1), (x2, o2), (x3, o3)):
        _agg_one_table(True, x_hbm, src_hbm, dst_hbm, zeros_hbm, out_hbm,
                       sidx, didx, rows0, rows1, sem0, sem1, acc,
                       c * _NS + s, c, s * _RPT)


def _make_sc_agg(do_gather):
    mesh = plsc.VectorSubcoreMesh(
        core_axis_name="c", subcore_axis_name="s", num_cores=_NC)
    return functools.partial(
        pl.kernel,
        mesh=mesh,
        out_type=jax.ShapeDtypeStruct((_NC * _NP, _D), jnp.float32),
        scratch_types=list(_SC_SCRATCH),
    )(functools.partial(_sc_agg_body, do_gather))


def _sc_agg(x, srcp, dstp, zeros_np):
    return _make_sc_agg(True)(x, srcp, dstp, zeros_np)


def _sc_deg(ones_ch, srcp, dstp, zeros_np):
    return _make_sc_agg(False)(ones_ch, srcp, dstp, zeros_np)


def _sc_aggx4(xs, srcp, dstp, zeros_np):
    mesh = plsc.VectorSubcoreMesh(
        core_axis_name="c", subcore_axis_name="s", num_cores=_NC)
    f = functools.partial(
        pl.kernel,
        mesh=mesh,
        out_type=[jax.ShapeDtypeStruct((_NC * _NP, _D), jnp.float32)] * 4,
        scratch_types=list(_SC_SCRATCH),
    )(_sc_aggx4_body)
    return f(xs[0], xs[1], xs[2], xs[3], srcp, dstp, zeros_np)


def _dinv_block(degp_ref):
    deg = degp_ref[0, :, 0:1] + degp_ref[1, :, 0:1]
    return 1.0 / jnp.maximum(deg, 1.0)


def _prep_body(x_ref, axp_ref, degp_ref, wxs_ref, wxn_ref, b_ref, p_ref):
    dinv = _dinv_block(degp_ref)
    aggx = (axp_ref[0, 0] + axp_ref[0, 1]) * dinv
    p_ref[0] = (jnp.dot(x_ref[0], wxs_ref[...],
                        preferred_element_type=jnp.float32)
                + jnp.dot(aggx, wxn_ref[...],
                          preferred_element_type=jnp.float32)
                + b_ref[...])


def _tc_prep(x, axp, degp, wxs, wxn, b):
    return pl.pallas_call(
        _prep_body,
        grid=(_T, _N // _R),
        in_specs=[
            pl.BlockSpec((1, _R, _D), lambda t, r: (t, r, 0)),
            pl.BlockSpec((1, 2, _R, _D), lambda t, r: (t, 0, r, 0)),
            pl.BlockSpec((2, _R, _D), lambda t, r: (0, r, 0)),
            pl.BlockSpec((_D, 2 * _D), lambda t, r: (0, 0)),
            pl.BlockSpec((_D, 2 * _D), lambda t, r: (0, 0)),
            pl.BlockSpec((1, 2 * _D), lambda t, r: (0, 0)),
        ],
        out_specs=pl.BlockSpec((1, _R, 2 * _D), lambda t, r: (t, r, 0)),
        out_shape=jax.ShapeDtypeStruct((_T, _N, 2 * _D), jnp.float32),
    )(x, axp, degp, wxs, wxn, b)


def _enc0_body(p_ref, h_ref):
    g = p_ref[...]
    u = jax.nn.sigmoid(g[:, :_D])
    cc = jnp.tanh(g[:, _D:])
    h_ref[...] = (1.0 - u) * cc


def _tc_enc0(p0):
    return pl.pallas_call(
        _enc0_body,
        grid=(_N // _R,),
        in_specs=[pl.BlockSpec((_R, 2 * _D), lambda r: (r, 0))],
        out_specs=pl.BlockSpec((_R, _D), lambda r: (r, 0)),
        out_shape=jax.ShapeDtypeStruct((_N, _D), jnp.float32),
    )(p0)


def _self_enc_body(p_ref, h_ref, whs_ref, gs_ref):
    gs_ref[...] = p_ref[...] + jnp.dot(h_ref[...], whs_ref[...],
                                       preferred_element_type=jnp.float32)


def _tc_self_enc(p, h, whs):
    return pl.pallas_call(
        _self_enc_body,
        grid=(_N // _R,),
        in_specs=[
            pl.BlockSpec((_R, 2 * _D), lambda r: (r, 0)),
            pl.BlockSpec((_R, _D), lambda r: (r, 0)),
            pl.BlockSpec((_D, 2 * _D), lambda r: (0, 0)),
        ],
        out_specs=pl.BlockSpec((_R, 2 * _D), lambda r: (r, 0)),
        out_shape=jax.ShapeDtypeStruct((_N, 2 * _D), jnp.float32),
    )(p, h, whs)


def _self_dec_body(h_ref, whs_ref, b_ref, gs_ref):
    gs_ref[...] = b_ref[...] + jnp.dot(h_ref[...], whs_ref[...],
                                       preferred_element_type=jnp.float32)


def _tc_self_dec(h, whs, b):
    return pl.pallas_call(
        _self_dec_body,
        grid=(_N // _R,),
        in_specs=[
            pl.BlockSpec((_R, _D), lambda r: (r, 0)),
            pl.BlockSpec((_D, 2 * _D), lambda r: (0, 0)),
            pl.BlockSpec((1, 2 * _D), lambda r: (0, 0)),
        ],
        out_specs=pl.BlockSpec((_R, 2 * _D), lambda r: (r, 0)),
        out_shape=jax.ShapeDtypeStruct((_N, 2 * _D), jnp.float32),
    )(h, whs, b)


def _step_body(gs_ref, h_ref, ahp_ref, degp_ref, whn_ref, hn_ref):
    dinv = _dinv_block(degp_ref)
    aggh = (ahp_ref[0] + ahp_ref[1]) * dinv
    h = h_ref[...]
    g = gs_ref[...] + jnp.dot(aggh, whn_ref[...],
                              preferred_element_type=jnp.float32)
    u = jax.nn.sigmoid(g[:, :_D])
    cc = jnp.tanh(g[:, _D:])
    hn_ref[...] = u * h + (1.0 - u) * cc


def _tc_step(gs, h, ahp, degp, whn):
    return pl.pallas_call(
        _step_body,
        grid=(_N // _R,),
        in_specs=[
            pl.BlockSpec((_R, 2 * _D), lambda r: (r, 0)),
            pl.BlockSpec((_R, _D), lambda r: (r, 0)),
            pl.BlockSpec((2, _R, _D), lambda r: (0, r, 0)),
            pl.BlockSpec((2, _R, _D), lambda r: (0, r, 0)),
            pl.BlockSpec((_D, 2 * _D), lambda r: (0, 0)),
        ],
        out_specs=pl.BlockSpec((_R, _D), lambda r: (r, 0)),
        out_shape=jax.ShapeDtypeStruct((_N, _D), jnp.float32),
    )(gs, h, ahp, degp, whn)


def _out_body(hs_ref, ow_ref, ob_ref, y_ref):
    y_ref[0] = (jnp.dot(hs_ref[0], ow_ref[...],
                        preferred_element_type=jnp.float32) + ob_ref[...])


def _tc_out(hs, ow, ob):
    return pl.pallas_call(
        _out_body,
        grid=(_T, _N // _R),
        in_specs=[
            pl.BlockSpec((1, _R, _D), lambda t, r: (t, r, 0)),
            pl.BlockSpec((_D, _D), lambda t, r: (0, 0)),
            pl.BlockSpec((1, _D), lambda t, r: (0, 0)),
        ],
        out_specs=pl.BlockSpec((1, _R, _D), lambda t, r: (t, r, 0)),
        out_shape=jax.ShapeDtypeStruct((_T, _N, _D), jnp.float32),
    )(hs, ow, ob)


def _stack_uc(w):
    return jnp.concatenate([w[1], w[2]], axis=1)


def _bias_uc(bx, bh):
    return jnp.concatenate([bx[1] + bh[1], bx[2] + bh[2]])[None, :]


def kernel(edge_index, inputs, teacher_states, batch_cnt,
           enc_Wx_self, enc_Wx_neigh, enc_bx, enc_Wh_self, enc_Wh_neigh,
           enc_bh, dec_Wx_self, dec_Wx_neigh, dec_bx, dec_Wh_self,
           dec_Wh_neigh, dec_bh, out_W, out_b):
    src = edge_index[0].astype(jnp.int32)
    dst = edge_index[1].astype(jnp.int32)
    npad = _EPAD - _E
    ar = jnp.arange(npad, dtype=jnp.int32)
    srcp = jnp.concatenate([src, (ar * 131) % _N]).reshape(
        _NC * _NS, _NCHUNK, _CH)
    dstp = jnp.concatenate([dst, _N + (ar % (_NP - _N))]).reshape(
        _NC * _NS, _NCHUNK, _CH)
    zeros_np = jnp.zeros((_NP, _D), jnp.float32)
    ones_ch = jnp.ones((_CH, _D), jnp.float32)

    degp = _sc_deg(ones_ch, srcp, dstp, zeros_np).reshape(_NC, _NP, _D)

    axp_list = _sc_aggx4([inputs[t] for t in range(_T)], srcp, dstp, zeros_np)
    axp = jnp.stack([a.reshape(_NC, _NP, _D) for a in axp_list])

    enc_wxs = _stack_uc(enc_Wx_self)
    enc_wxn = _stack_uc(enc_Wx_neigh)
    enc_whs = _stack_uc(enc_Wh_self)
    enc_whn = _stack_uc(enc_Wh_neigh)
    enc_b = _bias_uc(enc_bx, enc_bh)
    dec_whs = _stack_uc(dec_Wh_self)
    dec_whn = _stack_uc(dec_Wh_neigh)
    dec_b = _bias_uc(dec_bx, dec_bh)
    ow_t = out_W.T
    ob = out_b[None, :]

    p_all = _tc_prep(inputs, axp, degp, enc_wxs, enc_wxn, enc_b)

    h = _tc_enc0(p_all[0])
    for t in range(1, _T):
        ahp = _sc_agg(h, srcp, dstp, zeros_np).reshape(_NC, _NP, _D)
        gs = _tc_self_enc(p_all[t], h, enc_whs)
        h = _tc_step(gs, h, ahp, degp, enc_whn)

    hs = []
    for _ in range(_T):
        ahp = _sc_agg(h, srcp, dstp, zeros_np).reshape(_NC, _NP, _D)
        gs = _tc_self_dec(h, dec_whs, dec_b)
        h = _tc_step(gs, h, ahp, degp, dec_whn)
        hs.append(h)
    return _tc_out(jnp.stack(hs), ow_t, ob)


# speedup vs baseline: 1.0122x; 1.0122x over previous
import functools

import jax
import jax.numpy as jnp
from jax import lax
from jax.experimental import pallas as pl
from jax.experimental.pallas import tpu as pltpu
from jax.experimental.pallas import tpu_sc as plsc

_N = 10000
_E = 320000
_D = 128
_T = 4

_NC = 2
_NS = 16
_CH = 128
_NCHUNK = 80
_NPH = 2
_HC = _NCHUNK // _NPH
_EPT = _NCHUNK * _CH
_EPAD = _NC * _NS * _EPT
_NP = 10240
_RPT = _NP // _NS

_R = 1000


def _agg_one_table(do_gather, x_hbm, src_hbm, dst_hbm, zeros_hbm, out_hbm,
                   sidx, didx, rows0, rows1, sem0, sem1, acc, wid, c, r0):
    pltpu.sync_copy(zeros_hbm.at[pl.ds(r0, _RPT)], acc.at[pl.ds(r0, _RPT)])
    plsc.subcore_barrier()

    for p in range(_NPH):
        if do_gather:
            pltpu.sync_copy(src_hbm.at[wid, pl.ds(p * _HC, _HC)], sidx)
        pltpu.sync_copy(dst_hbm.at[wid, pl.ds(p * _HC, _HC)], didx)

        if do_gather:
            def fire(j, rows, sem):
                pltpu.async_copy(x_hbm.at[sidx.at[j]], rows, sem)

            def drain(rows, sem):
                pltpu.make_async_copy(x_hbm.at[sidx.at[0]], rows, sem).wait()

            fire(0, rows0, sem0)
            fire(1, rows1, sem1)

            def group(gi, carry):
                drain(rows0, sem0)
                pltpu.sync_copy(rows0, acc.at[didx.at[2 * gi]], add=True)
                fire(2 * gi + 2, rows0, sem0)
                drain(rows1, sem1)
                pltpu.sync_copy(rows1, acc.at[didx.at[2 * gi + 1]], add=True)
                fire(2 * gi + 3, rows1, sem1)
                return carry

            lax.fori_loop(0, _HC // 2 - 1, group, 0)
            drain(rows0, sem0)
            pltpu.sync_copy(rows0, acc.at[didx.at[_HC - 2]], add=True)
            drain(rows1, sem1)
            pltpu.sync_copy(rows1, acc.at[didx.at[_HC - 1]], add=True)
        else:
            def chunk(j, carry):
                pltpu.sync_copy(rows0, acc.at[didx.at[j]], add=True)
                return carry

            lax.fori_loop(0, _HC, chunk, 0)

    plsc.subcore_barrier()
    pltpu.sync_copy(acc.at[pl.ds(r0, _RPT)],
                    out_hbm.at[pl.ds(c * _NP + r0, _RPT)])


_SC_SCRATCH = [
    pltpu.VMEM((_HC, _CH), jnp.int32),
    pltpu.VMEM((_HC, _CH), jnp.int32),
    pltpu.VMEM((_CH, _D), jnp.float32),
    pltpu.VMEM((_CH, _D), jnp.float32),
    pltpu.SemaphoreType.DMA,
    pltpu.SemaphoreType.DMA,
    pltpu.VMEM_SHARED((_NP, _D), jnp.float32),
]


def _sc_agg_body(x_hbm, src_hbm, dst_hbm, zeros_hbm, out_hbm,
                 sidx, didx, rows0, rows1, sem0, sem1, acc):
    c = lax.axis_index("c")
    s = lax.axis_index("s")
    _agg_one_table(True, x_hbm, src_hbm, dst_hbm, zeros_hbm, out_hbm,
                   sidx, didx, rows0, rows1, sem0, sem1, acc,
                   c * _NS + s, c, s * _RPT)


def _sc_deg_body(x_hbm, src_hbm, dst_hbm, zeros_hbm, out_hbm,
                 sidx, didx, rows0, rows1, sem0, sem1, acc):
    c = lax.axis_index("c")
    s = lax.axis_index("s")
    pltpu.sync_copy(x_hbm, rows0)
    _agg_one_table(False, x_hbm, src_hbm, dst_hbm, zeros_hbm, out_hbm,
                   sidx, didx, rows0, rows1, sem0, sem1, acc,
                   c * _NS + s, c, s * _RPT)


def _sc_aggx4_body(x0, x1, x2, x3, src_hbm, dst_hbm, zeros_hbm,
                   o0, o1, o2, o3, sidx, didx, rows0, rows1, sem0, sem1, acc):
    c = lax.axis_index("c")
    s = lax.axis_index("s")
    for x_hbm, out_hbm in ((x0, o0), (x1, o1), (x2, o2), (x3, o3)):
        _agg_one_table(True, x_hbm, src_hbm, dst_hbm, zeros_hbm, out_hbm,
                       sidx, didx, rows0, rows1, sem0, sem1, acc,
                       c * _NS + s, c, s * _RPT)


def _mesh():
    return plsc.VectorSubcoreMesh(
        core_axis_name="c", subcore_axis_name="s", num_cores=_NC)


def _sc_agg(x, srcp, dstp, zeros_np):
    f = functools.partial(
        pl.kernel,
        mesh=_mesh(),
        out_type=jax.ShapeDtypeStruct((_NC * _NP, _D), jnp.float32),
        scratch_types=list(_SC_SCRATCH),
    )(_sc_agg_body)
    return f(x, srcp, dstp, zeros_np)


def _sc_deg(ones_ch, srcp, dstp, zeros_np):
    f = functools.partial(
        pl.kernel,
        mesh=_mesh(),
        out_type=jax.ShapeDtypeStruct((_NC * _NP, _D), jnp.float32),
        scratch_types=list(_SC_SCRATCH),
    )(_sc_deg_body)
    return f(ones_ch, srcp, dstp, zeros_np)


def _sc_aggx4(xs, srcp, dstp, zeros_np):
    mesh = plsc.VectorSubcoreMesh(
        core_axis_name="c", subcore_axis_name="s", num_cores=_NC)
    f = functools.partial(
        pl.kernel,
        mesh=mesh,
        out_type=[jax.ShapeDtypeStruct((_NC * _NP, _D), jnp.float32)] * 4,
        scratch_types=list(_SC_SCRATCH),
    )(_sc_aggx4_body)
    return f(xs[0], xs[1], xs[2], xs[3], srcp, dstp, zeros_np)


def _dinv_block(degp_ref):
    deg = degp_ref[0, :, 0:1] + degp_ref[1, :, 0:1]
    return 1.0 / jnp.maximum(deg, 1.0)


def _prep_body(x_ref, axp_ref, degp_ref, wxs_ref, wxn_ref, b_ref, p_ref):
    dinv = _dinv_block(degp_ref)
    aggx = (axp_ref[0, 0] + axp_ref[0, 1]) * dinv
    p_ref[0] = (jnp.dot(x_ref[0], wxs_ref[...],
                        preferred_element_type=jnp.float32)
                + jnp.dot(aggx, wxn_ref[...],
                          preferred_element_type=jnp.float32)
                + b_ref[...])


def _tc_prep(x, axp, degp, wxs, wxn, b):
    return pl.pallas_call(
        _prep_body,
        grid=(_T, _N // _R),
        in_specs=[
            pl.BlockSpec((1, _R, _D), lambda t, r: (t, r, 0)),
            pl.BlockSpec((1, 2, _R, _D), lambda t, r: (t, 0, r, 0)),
            pl.BlockSpec((2, _R, _D), lambda t, r: (0, r, 0)),
            pl.BlockSpec((_D, 2 * _D), lambda t, r: (0, 0)),
            pl.BlockSpec((_D, 2 * _D), lambda t, r: (0, 0)),
            pl.BlockSpec((1, 2 * _D), lambda t, r: (0, 0)),
        ],
        out_specs=pl.BlockSpec((1, _R, 2 * _D), lambda t, r: (t, r, 0)),
        out_shape=jax.ShapeDtypeStruct((_T, _N, 2 * _D), jnp.float32),
    )(x, axp, degp, wxs, wxn, b)


def _enc0_body(p_ref, h_ref):
    g = p_ref[...]
    u = jax.nn.sigmoid(g[:, :_D])
    cc = jnp.tanh(g[:, _D:])
    h_ref[...] = (1.0 - u) * cc


def _tc_enc0(p0):
    return pl.pallas_call(
        _enc0_body,
        grid=(_N // _R,),
        in_specs=[pl.BlockSpec((_R, 2 * _D), lambda r: (r, 0))],
        out_specs=pl.BlockSpec((_R, _D), lambda r: (r, 0)),
        out_shape=jax.ShapeDtypeStruct((_N, _D), jnp.float32),
    )(p0)


def _enc_body(p_ref, h_ref, ahp_ref, degp_ref, whs_ref, whn_ref, hn_ref):
    dinv = _dinv_block(degp_ref)
    aggh = (ahp_ref[0] + ahp_ref[1]) * dinv
    h = h_ref[...]
    g = (p_ref[...]
         + jnp.dot(h, whs_ref[...], preferred_element_type=jnp.float32)
         + jnp.dot(aggh, whn_ref[...], preferred_element_type=jnp.float32))
    u = jax.nn.sigmoid(g[:, :_D])
    cc = jnp.tanh(g[:, _D:])
    hn_ref[...] = u * h + (1.0 - u) * cc


def _tc_enc(p, h, ahp, degp, whs, whn):
    return pl.pallas_call(
        _enc_body,
        grid=(_N // _R,),
        in_specs=[
            pl.BlockSpec((_R, 2 * _D), lambda r: (r, 0)),
            pl.BlockSpec((_R, _D), lambda r: (r, 0)),
            pl.BlockSpec((2, _R, _D), lambda r: (0, r, 0)),
            pl.BlockSpec((2, _R, _D), lambda r: (0, r, 0)),
            pl.BlockSpec((_D, 2 * _D), lambda r: (0, 0)),
            pl.BlockSpec((_D, 2 * _D), lambda r: (0, 0)),
        ],
        out_specs=pl.BlockSpec((_R, _D), lambda r: (r, 0)),
        out_shape=jax.ShapeDtypeStruct((_N, _D), jnp.float32),
    )(p, h, ahp, degp, whs, whn)


def _dec_body(h_ref, ahp_ref, degp_ref, whs_ref, whn_ref, b_ref, hn_ref):
    dinv = _dinv_block(degp_ref)
    aggh = (ahp_ref[0] + ahp_ref[1]) * dinv
    h = h_ref[...]
    g = (jnp.dot(h, whs_ref[...], preferred_element_type=jnp.float32)
         + jnp.dot(aggh, whn_ref[...], preferred_element_type=jnp.float32)
         + b_ref[...])
    u = jax.nn.sigmoid(g[:, :_D])
    cc = jnp.tanh(g[:, _D:])
    hn_ref[...] = u * h + (1.0 - u) * cc


def _tc_dec(h, ahp, degp, whs, whn, b):
    return pl.pallas_call(
        _dec_body,
        grid=(_N // _R,),
        in_specs=[
            pl.BlockSpec((_R, _D), lambda r: (r, 0)),
            pl.BlockSpec((2, _R, _D), lambda r: (0, r, 0)),
            pl.BlockSpec((2, _R, _D), lambda r: (0, r, 0)),
            pl.BlockSpec((_D, 2 * _D), lambda r: (0, 0)),
            pl.BlockSpec((_D, 2 * _D), lambda r: (0, 0)),
            pl.BlockSpec((1, 2 * _D), lambda r: (0, 0)),
        ],
        out_specs=pl.BlockSpec((_R, _D), lambda r: (r, 0)),
        out_shape=jax.ShapeDtypeStruct((_N, _D), jnp.float32),
    )(h, ahp, degp, whs, whn, b)


def _out_body(hs_ref, ow_ref, ob_ref, y_ref):
    y_ref[0] = (jnp.dot(hs_ref[0], ow_ref[...],
                        preferred_element_type=jnp.float32) + ob_ref[...])


def _tc_out(hs, ow, ob):
    return pl.pallas_call(
        _out_body,
        grid=(_T, _N // _R),
        in_specs=[
            pl.BlockSpec((1, _R, _D), lambda t, r: (t, r, 0)),
            pl.BlockSpec((_D, _D), lambda t, r: (0, 0)),
            pl.BlockSpec((1, _D), lambda t, r: (0, 0)),
        ],
        out_specs=pl.BlockSpec((1, _R, _D), lambda t, r: (t, r, 0)),
        out_shape=jax.ShapeDtypeStruct((_T, _N, _D), jnp.float32),
    )(hs, ow, ob)


def _stack_uc(w):
    return jnp.concatenate([w[1], w[2]], axis=1)


def _bias_uc(bx, bh):
    return jnp.concatenate([bx[1] + bh[1], bx[2] + bh[2]])[None, :]


def kernel(edge_index, inputs, teacher_states, batch_cnt,
           enc_Wx_self, enc_Wx_neigh, enc_bx, enc_Wh_self, enc_Wh_neigh,
           enc_bh, dec_Wx_self, dec_Wx_neigh, dec_bx, dec_Wh_self,
           dec_Wh_neigh, dec_bh, out_W, out_b):
    src = edge_index[0].astype(jnp.int32)
    dst = edge_index[1].astype(jnp.int32)
    npad = _EPAD - _E
    ar = jnp.arange(npad, dtype=jnp.int32)
    srcp = jnp.concatenate([src, (ar * 131) % _N]).reshape(
        _NC * _NS, _NCHUNK, _CH)
    dstp = jnp.concatenate([dst, _N + (ar % (_NP - _N))]).reshape(
        _NC * _NS, _NCHUNK, _CH)
    zeros_np = jnp.zeros((_NP, _D), jnp.float32)
    ones_ch = jnp.ones((_CH, _D), jnp.float32)

    degp = _sc_deg(ones_ch, srcp, dstp, zeros_np).reshape(_NC, _NP, _D)

    axp_list = _sc_aggx4([inputs[t] for t in range(_T)], srcp, dstp, zeros_np)
    axp = jnp.stack([a.reshape(_NC, _NP, _D) for a in axp_list])

    enc_wxs = _stack_uc(enc_Wx_self)
    enc_wxn = _stack_uc(enc_Wx_neigh)
    enc_whs = _stack_uc(enc_Wh_self)
    enc_whn = _stack_uc(enc_Wh_neigh)
    enc_b = _bias_uc(enc_bx, enc_bh)
    dec_whs = _stack_uc(dec_Wh_self)
    dec_whn = _stack_uc(dec_Wh_neigh)
    dec_b = _bias_uc(dec_bx, dec_bh)
    ow_t = out_W.T
    ob = out_b[None, :]

    p_all = _tc_prep(inputs, axp, degp, enc_wxs, enc_wxn, enc_b)

    h = _tc_enc0(p_all[0])
    for t in range(1, _T):
        ahp = _sc_agg(h, srcp, dstp, zeros_np).reshape(_NC, _NP, _D)
        h = _tc_enc(p_all[t], h, ahp, degp, enc_whs, enc_whn)

    hs = []
    for _ in range(_T):
        ahp = _sc_agg(h, srcp, dstp, zeros_np).reshape(_NC, _NP, _D)
        h = _tc_dec(h, ahp, degp, dec_whs, dec_whn, dec_b)
        hs.append(h)
    return _tc_out(jnp.stack(hs), ow_t, ob)


# speedup vs baseline: 1.0418x; 1.0292x over previous
import functools

import jax
import jax.numpy as jnp
from jax import lax
from jax.experimental import pallas as pl
from jax.experimental.pallas import tpu as pltpu
from jax.experimental.pallas import tpu_sc as plsc

_N = 10000
_E = 320000
_D = 128
_T = 4

_NC = 2
_NS = 16
_CH = 128
_NCHUNK = 80
_NPH = 2
_HC = _NCHUNK // _NPH
_EPT = _NCHUNK * _CH
_EPAD = _NC * _NS * _EPT
_NP = 10240
_RPT = _NP // _NS

_R = 1000


def _agg_one_table(do_gather, x_hbm, src_hbm, dst_hbm, zeros_hbm, out_hbm,
                   sidx, didx, rows0, rows1, sem0, sem1, acc, wid, c, r0):
    pltpu.sync_copy(zeros_hbm.at[pl.ds(r0, _RPT)], acc.at[pl.ds(r0, _RPT)])
    plsc.subcore_barrier()

    for p in range(_NPH):
        if do_gather:
            pltpu.sync_copy(src_hbm.at[wid, pl.ds(p * _HC, _HC)], sidx)
        pltpu.sync_copy(dst_hbm.at[wid, pl.ds(p * _HC, _HC)], didx)

        if do_gather:
            def fire(j, rows, sem):
                pltpu.async_copy(x_hbm.at[sidx.at[j]], rows, sem)

            def drain(rows, sem):
                pltpu.make_async_copy(x_hbm.at[sidx.at[0]], rows, sem).wait()

            fire(0, rows0, sem0)
            fire(1, rows1, sem1)

            def group(gi, carry):
                drain(rows0, sem0)
                pltpu.sync_copy(rows0, acc.at[didx.at[2 * gi]], add=True)
                fire(2 * gi + 2, rows0, sem0)
                drain(rows1, sem1)
                pltpu.sync_copy(rows1, acc.at[didx.at[2 * gi + 1]], add=True)
                fire(2 * gi + 3, rows1, sem1)
                return carry

            lax.fori_loop(0, _HC // 2 - 1, group, 0)
            drain(rows0, sem0)
            pltpu.sync_copy(rows0, acc.at[didx.at[_HC - 2]], add=True)
            drain(rows1, sem1)
            pltpu.sync_copy(rows1, acc.at[didx.at[_HC - 1]], add=True)
        else:
            def chunk(j, carry):
                pltpu.sync_copy(rows0, acc.at[didx.at[j]], add=True)
                return carry

            lax.fori_loop(0, _HC, chunk, 0)

    plsc.subcore_barrier()
    pltpu.sync_copy(acc.at[pl.ds(r0, _RPT)],
                    out_hbm.at[pl.ds(c * _NP + r0, _RPT)])


_SC_SCRATCH = [
    pltpu.VMEM((_HC, _CH), jnp.int32),
    pltpu.VMEM((_HC, _CH), jnp.int32),
    pltpu.VMEM((_CH, _D), jnp.float32),
    pltpu.VMEM((_CH, _D), jnp.float32),
    pltpu.SemaphoreType.DMA,
    pltpu.SemaphoreType.DMA,
    pltpu.VMEM_SHARED((_NP, _D), jnp.float32),
]


def _sc_agg_body(x_hbm, src_hbm, dst_hbm, zeros_hbm, out_hbm,
                 sidx, didx, rows0, rows1, sem0, sem1, acc):
    c = lax.axis_index("c")
    s = lax.axis_index("s")
    _agg_one_table(True, x_hbm, src_hbm, dst_hbm, zeros_hbm, out_hbm,
                   sidx, didx, rows0, rows1, sem0, sem1, acc,
                   c * _NS + s, c, s * _RPT)


def _sc_deg_body(dst_hbm, out_hbm, didx, ones1, z1, acc1):
    c = lax.axis_index("c")
    s = lax.axis_index("s")
    wid = c * _NS + s
    r0 = s * _RPT
    for i in range(_CH // 16):
        ones1[pl.ds(16 * i, 16)] = jnp.ones((16,), jnp.float32)
        z1[pl.ds(16 * i, 16)] = jnp.zeros((16,), jnp.float32)
    for q in range(_RPT // _CH):
        pltpu.sync_copy(z1, acc1.at[pl.ds(r0 + q * _CH, _CH)])
    plsc.subcore_barrier()
    for p in range(_NPH):
        pltpu.sync_copy(dst_hbm.at[wid, pl.ds(p * _HC, _HC)], didx)

        def chunk(j, carry):
            pltpu.sync_copy(ones1, acc1.at[didx.at[j]], add=True)
            return carry

        lax.fori_loop(0, _HC, chunk, 0)
    plsc.subcore_barrier()
    pltpu.sync_copy(acc1.at[pl.ds(r0, _RPT)],
                    out_hbm.at[pl.ds(c * _NP + r0, _RPT)])


def _sc_aggx4_body(x0, x1, x2, x3, src_hbm, dst_hbm, zeros_hbm,
                   o0, o1, o2, o3, sidx, didx, rows0, rows1, sem0, sem1, acc):
    c = lax.axis_index("c")
    s = lax.axis_index("s")
    for x_hbm, out_hbm in ((x0, o0), (x1, o1), (x2, o2), (x3, o3)):
        _agg_one_table(True, x_hbm, src_hbm, dst_hbm, zeros_hbm, out_hbm,
                       sidx, didx, rows0, rows1, sem0, sem1, acc,
                       c * _NS + s, c, s * _RPT)


def _mesh():
    return plsc.VectorSubcoreMesh(
        core_axis_name="c", subcore_axis_name="s", num_cores=_NC)


def _sc_agg(x, srcp, dstp, zeros_np):
    f = functools.partial(
        pl.kernel,
        mesh=_mesh(),
        out_type=jax.ShapeDtypeStruct((_NC * _NP, _D), jnp.float32),
        scratch_types=list(_SC_SCRATCH),
    )(_sc_agg_body)
    return f(x, srcp, dstp, zeros_np)


def _sc_deg(dstp):
    f = functools.partial(
        pl.kernel,
        mesh=_mesh(),
        out_type=jax.ShapeDtypeStruct((_NC * _NP,), jnp.float32),
        scratch_types=[
            pltpu.VMEM((_HC, _CH), jnp.int32),
            pltpu.VMEM((_CH,), jnp.float32),
            pltpu.VMEM((_CH,), jnp.float32),
            pltpu.VMEM_SHARED((_NP,), jnp.float32),
        ],
    )(_sc_deg_body)
    return f(dstp)


def _sc_aggx4(xs, srcp, dstp, zeros_np):
    mesh = plsc.VectorSubcoreMesh(
        core_axis_name="c", subcore_axis_name="s", num_cores=_NC)
    f = functools.partial(
        pl.kernel,
        mesh=mesh,
        out_type=[jax.ShapeDtypeStruct((_NC * _NP, _D), jnp.float32)] * 4,
        scratch_types=list(_SC_SCRATCH),
    )(_sc_aggx4_body)
    return f(xs[0], xs[1], xs[2], xs[3], srcp, dstp, zeros_np)


def _dinv_block(degp_ref):
    deg = degp_ref[0, :, 0:1] + degp_ref[1, :, 0:1]
    return 1.0 / jnp.maximum(deg, 1.0)


def _prep_body(x_ref, axp_ref, degp_ref, wxs_ref, wxn_ref, b_ref, p_ref):
    dinv = _dinv_block(degp_ref)
    aggx = (axp_ref[0, 0] + axp_ref[0, 1]) * dinv
    p_ref[0] = (jnp.dot(x_ref[0], wxs_ref[...],
                        preferred_element_type=jnp.float32)
                + jnp.dot(aggx, wxn_ref[...],
                          preferred_element_type=jnp.float32)
                + b_ref[...])


def _tc_prep(x, axp, degp, wxs, wxn, b):
    return pl.pallas_call(
        _prep_body,
        grid=(_T, _N // _R),
        in_specs=[
            pl.BlockSpec((1, _R, _D), lambda t, r: (t, r, 0)),
            pl.BlockSpec((1, 2, _R, _D), lambda t, r: (t, 0, r, 0)),
            pl.BlockSpec((2, _R, 1), lambda t, r: (0, r, 0)),
            pl.BlockSpec((_D, 2 * _D), lambda t, r: (0, 0)),
            pl.BlockSpec((_D, 2 * _D), lambda t, r: (0, 0)),
            pl.BlockSpec((1, 2 * _D), lambda t, r: (0, 0)),
        ],
        out_specs=pl.BlockSpec((1, _R, 2 * _D), lambda t, r: (t, r, 0)),
        out_shape=jax.ShapeDtypeStruct((_T, _N, 2 * _D), jnp.float32),
    )(x, axp, degp, wxs, wxn, b)


def _enc0_body(p_ref, h_ref):
    g = p_ref[...]
    u = jax.nn.sigmoid(g[:, :_D])
    cc = jnp.tanh(g[:, _D:])
    h_ref[...] = (1.0 - u) * cc


def _tc_enc0(p0):
    return pl.pallas_call(
        _enc0_body,
        grid=(_N // _R,),
        in_specs=[pl.BlockSpec((_R, 2 * _D), lambda r: (r, 0))],
        out_specs=pl.BlockSpec((_R, _D), lambda r: (r, 0)),
        out_shape=jax.ShapeDtypeStruct((_N, _D), jnp.float32),
    )(p0)


def _enc_body(p_ref, h_ref, ahp_ref, degp_ref, whs_ref, whn_ref, hn_ref):
    dinv = _dinv_block(degp_ref)
    aggh = (ahp_ref[0] + ahp_ref[1]) * dinv
    h = h_ref[...]
    g = (p_ref[...]
         + jnp.dot(h, whs_ref[...], preferred_element_type=jnp.float32)
         + jnp.dot(aggh, whn_ref[...], preferred_element_type=jnp.float32))
    u = jax.nn.sigmoid(g[:, :_D])
    cc = jnp.tanh(g[:, _D:])
    hn_ref[...] = u * h + (1.0 - u) * cc


def _tc_enc(p, h, ahp, degp, whs, whn):
    return pl.pallas_call(
        _enc_body,
        grid=(_N // _R,),
        in_specs=[
            pl.BlockSpec((_R, 2 * _D), lambda r: (r, 0)),
            pl.BlockSpec((_R, _D), lambda r: (r, 0)),
            pl.BlockSpec((2, _R, _D), lambda r: (0, r, 0)),
            pl.BlockSpec((2, _R, 1), lambda r: (0, r, 0)),
            pl.BlockSpec((_D, 2 * _D), lambda r: (0, 0)),
            pl.BlockSpec((_D, 2 * _D), lambda r: (0, 0)),
        ],
        out_specs=pl.BlockSpec((_R, _D), lambda r: (r, 0)),
        out_shape=jax.ShapeDtypeStruct((_N, _D), jnp.float32),
    )(p, h, ahp, degp, whs, whn)


def _dec_body(h_ref, ahp_ref, degp_ref, whs_ref, whn_ref, b_ref, hn_ref):
    dinv = _dinv_block(degp_ref)
    aggh = (ahp_ref[0] + ahp_ref[1]) * dinv
    h = h_ref[...]
    g = (jnp.dot(h, whs_ref[...], preferred_element_type=jnp.float32)
         + jnp.dot(aggh, whn_ref[...], preferred_element_type=jnp.float32)
         + b_ref[...])
    u = jax.nn.sigmoid(g[:, :_D])
    cc = jnp.tanh(g[:, _D:])
    hn_ref[...] = u * h + (1.0 - u) * cc


def _tc_dec(h, ahp, degp, whs, whn, b):
    return pl.pallas_call(
        _dec_body,
        grid=(_N // _R,),
        in_specs=[
            pl.BlockSpec((_R, _D), lambda r: (r, 0)),
            pl.BlockSpec((2, _R, _D), lambda r: (0, r, 0)),
            pl.BlockSpec((2, _R, 1), lambda r: (0, r, 0)),
            pl.BlockSpec((_D, 2 * _D), lambda r: (0, 0)),
            pl.BlockSpec((_D, 2 * _D), lambda r: (0, 0)),
            pl.BlockSpec((1, 2 * _D), lambda r: (0, 0)),
        ],
        out_specs=pl.BlockSpec((_R, _D), lambda r: (r, 0)),
        out_shape=jax.ShapeDtypeStruct((_N, _D), jnp.float32),
    )(h, ahp, degp, whs, whn, b)


def _out_body(hs_ref, ow_ref, ob_ref, y_ref):
    y_ref[0] = (jnp.dot(hs_ref[0], ow_ref[...],
                        preferred_element_type=jnp.float32) + ob_ref[...])


def _tc_out(hs, ow, ob):
    return pl.pallas_call(
        _out_body,
        grid=(_T, _N // _R),
        in_specs=[
            pl.BlockSpec((1, _R, _D), lambda t, r: (t, r, 0)),
            pl.BlockSpec((_D, _D), lambda t, r: (0, 0)),
            pl.BlockSpec((1, _D), lambda t, r: (0, 0)),
        ],
        out_specs=pl.BlockSpec((1, _R, _D), lambda t, r: (t, r, 0)),
        out_shape=jax.ShapeDtypeStruct((_T, _N, _D), jnp.float32),
    )(hs, ow, ob)


def _stack_uc(w):
    return jnp.concatenate([w[1], w[2]], axis=1)


def _bias_uc(bx, bh):
    return jnp.concatenate([bx[1] + bh[1], bx[2] + bh[2]])[None, :]


def kernel(edge_index, inputs, teacher_states, batch_cnt,
           enc_Wx_self, enc_Wx_neigh, enc_bx, enc_Wh_self, enc_Wh_neigh,
           enc_bh, dec_Wx_self, dec_Wx_neigh, dec_bx, dec_Wh_self,
           dec_Wh_neigh, dec_bh, out_W, out_b):
    src = edge_index[0].astype(jnp.int32)
    dst = edge_index[1].astype(jnp.int32)
    npad = _EPAD - _E
    ar = jnp.arange(npad, dtype=jnp.int32)
    srcp = jnp.concatenate([src, (ar * 131) % _N]).reshape(
        _NC * _NS, _NCHUNK, _CH)
    dstp = jnp.concatenate([dst, _N + (ar % (_NP - _N))]).reshape(
        _NC * _NS, _NCHUNK, _CH)
    zeros_np = jnp.zeros((_NP, _D), jnp.float32)

    degp = _sc_deg(dstp).reshape(_NC, _NP, 1)

    axp_list = _sc_aggx4([inputs[t] for t in range(_T)], srcp, dstp, zeros_np)
    axp = jnp.stack([a.reshape(_NC, _NP, _D) for a in axp_list])

    enc_wxs = _stack_uc(enc_Wx_self)
    enc_wxn = _stack_uc(enc_Wx_neigh)
    enc_whs = _stack_uc(enc_Wh_self)
    enc_whn = _stack_uc(enc_Wh_neigh)
    enc_b = _bias_uc(enc_bx, enc_bh)
    dec_whs = _stack_uc(dec_Wh_self)
    dec_whn = _stack_uc(dec_Wh_neigh)
    dec_b = _bias_uc(dec_bx, dec_bh)
    ow_t = out_W.T
    ob = out_b[None, :]

    p_all = _tc_prep(inputs, axp, degp, enc_wxs, enc_wxn, enc_b)

    h = _tc_enc0(p_all[0])
    for t in range(1, _T):
        ahp = _sc_agg(h, srcp, dstp, zeros_np).reshape(_NC, _NP, _D)
        h = _tc_enc(p_all[t], h, ahp, degp, enc_whs, enc_whn)

    hs = []
    for _ in range(_T):
        ahp = _sc_agg(h, srcp, dstp, zeros_np).reshape(_NC, _NP, _D)
        h = _tc_dec(h, ahp, degp, dec_whs, dec_whn, dec_b)
        hs.append(h)
    return _tc_out(jnp.stack(hs), ow_t, ob)


# speedup vs baseline: 1.0456x; 1.0037x over previous
import functools

import jax
import jax.numpy as jnp
from jax import lax
from jax.experimental import pallas as pl
from jax.experimental.pallas import tpu as pltpu
from jax.experimental.pallas import tpu_sc as plsc

_N = 10000
_E = 320000
_D = 128
_T = 4

_NC = 2
_NS = 16
_CH = 128
_NCHUNK = 80
_NPH = 2
_HC = _NCHUNK // _NPH
_EPT = _NCHUNK * _CH
_EPAD = _NC * _NS * _EPT
_NP = 10240
_RPT = _NP // _NS

_R = 1000


def _agg_one_table(x_hbm, src_hbm, dst_hbm, zeros_hbm, out_hbm,
                   sidx, didx, rows0, rows1, sem0, sem1, acc, wid, c, r0):
    pltpu.sync_copy(zeros_hbm.at[pl.ds(r0, _RPT)], acc.at[pl.ds(r0, _RPT)])
    plsc.subcore_barrier()

    for p in range(_NPH):
        pltpu.sync_copy(src_hbm.at[wid, pl.ds(p * _HC, _HC)], sidx)
        pltpu.sync_copy(dst_hbm.at[wid, pl.ds(p * _HC, _HC)], didx)

        def fire(j, rows, sem):
            pltpu.async_copy(x_hbm.at[sidx.at[j]], rows, sem)

        def drain(rows, sem):
            pltpu.make_async_copy(x_hbm.at[sidx.at[0]], rows, sem).wait()

        fire(0, rows0, sem0)
        fire(1, rows1, sem1)

        def group(gi, carry):
            drain(rows0, sem0)
            pltpu.sync_copy(rows0, acc.at[didx.at[2 * gi]], add=True)
            fire(2 * gi + 2, rows0, sem0)
            drain(rows1, sem1)
            pltpu.sync_copy(rows1, acc.at[didx.at[2 * gi + 1]], add=True)
            fire(2 * gi + 3, rows1, sem1)
            return carry

        lax.fori_loop(0, _HC // 2 - 1, group, 0)
        drain(rows0, sem0)
        pltpu.sync_copy(rows0, acc.at[didx.at[_HC - 2]], add=True)
        drain(rows1, sem1)
        pltpu.sync_copy(rows1, acc.at[didx.at[_HC - 1]], add=True)

    plsc.subcore_barrier()
    pltpu.sync_copy(acc.at[pl.ds(r0, _RPT)],
                    out_hbm.at[pl.ds(c * _NP + r0, _RPT)])


_SC_SCRATCH = [
    pltpu.VMEM((_HC, _CH), jnp.int32),
    pltpu.VMEM((_HC, _CH), jnp.int32),
    pltpu.VMEM((_CH, _D), jnp.float32),
    pltpu.VMEM((_CH, _D), jnp.float32),
    pltpu.SemaphoreType.DMA,
    pltpu.SemaphoreType.DMA,
    pltpu.VMEM_SHARED((_NP, _D), jnp.float32),
]


def _sc_agg_body(x_hbm, src_hbm, dst_hbm, zeros_hbm, out_hbm,
                 sidx, didx, rows0, rows1, sem0, sem1, acc):
    c = lax.axis_index("c")
    s = lax.axis_index("s")
    _agg_one_table(x_hbm, src_hbm, dst_hbm, zeros_hbm, out_hbm,
                   sidx, didx, rows0, rows1, sem0, sem1, acc,
                   c * _NS + s, c, s * _RPT)


def _sc_deg_body(dst_hbm, out_hbm, didx, ones1, z1, acc1):
    c = lax.axis_index("c")
    s = lax.axis_index("s")
    wid = c * _NS + s
    r0 = s * _RPT
    for i in range(_CH // 16):
        ones1[pl.ds(16 * i, 16)] = jnp.ones((16,), jnp.float32)
        z1[pl.ds(16 * i, 16)] = jnp.zeros((16,), jnp.float32)
    for q in range(_RPT // _CH):
        pltpu.sync_copy(z1, acc1.at[pl.ds(r0 + q * _CH, _CH)])
    plsc.subcore_barrier()
    for p in range(_NPH):
        pltpu.sync_copy(dst_hbm.at[wid, pl.ds(p * _HC, _HC)], didx)

        def chunk(j, carry):
            pltpu.sync_copy(ones1, acc1.at[didx.at[j]], add=True)
            return carry

        lax.fori_loop(0, _HC, chunk, 0)
    plsc.subcore_barrier()
    pltpu.sync_copy(acc1.at[pl.ds(r0, _RPT)],
                    out_hbm.at[pl.ds(c * _NP + r0, _RPT)])


def _sc_aggx4_body(x0, x1, x2, x3, src_hbm, dst_hbm, zeros_hbm,
                   o0, o1, o2, o3, sidx, didx, rows0, rows1, sem0, sem1, acc):
    c = lax.axis_index("c")
    s = lax.axis_index("s")
    for x_hbm, out_hbm in ((x0, o0), (x1, o1), (x2, o2), (x3, o3)):
        _agg_one_table(x_hbm, src_hbm, dst_hbm, zeros_hbm, out_hbm,
                       sidx, didx, rows0, rows1, sem0, sem1, acc,
                       c * _NS + s, c, s * _RPT)


def _mesh():
    return plsc.VectorSubcoreMesh(
        core_axis_name="c", subcore_axis_name="s", num_cores=_NC)


def _sc_agg(x, srcp, dstp, zeros_np):
    f = functools.partial(
        pl.kernel,
        mesh=_mesh(),
        out_type=jax.ShapeDtypeStruct((_NC * _NP, _D), jnp.float32),
        scratch_types=list(_SC_SCRATCH),
    )(_sc_agg_body)
    return f(x, srcp, dstp, zeros_np)


def _sc_deg(dstp):
    f = functools.partial(
        pl.kernel,
        mesh=_mesh(),
        out_type=jax.ShapeDtypeStruct((_NC * _NP,), jnp.float32),
        scratch_types=[
            pltpu.VMEM((_HC, _CH), jnp.int32),
            pltpu.VMEM((_CH,), jnp.float32),
            pltpu.VMEM((_CH,), jnp.float32),
            pltpu.VMEM_SHARED((_NP,), jnp.float32),
        ],
    )(_sc_deg_body)
    return f(dstp)


def _sc_aggx4(xs, srcp, dstp, zeros_np):
    mesh = plsc.VectorSubcoreMesh(
        core_axis_name="c", subcore_axis_name="s", num_cores=_NC)
    f = functools.partial(
        pl.kernel,
        mesh=mesh,
        out_type=[jax.ShapeDtypeStruct((_NC * _NP, _D), jnp.float32)] * 4,
        scratch_types=list(_SC_SCRATCH),
    )(_sc_aggx4_body)
    return f(xs[0], xs[1], xs[2], xs[3], srcp, dstp, zeros_np)


def _dinv_block(degp_ref):
    deg = degp_ref[0, :, 0:1] + degp_ref[1, :, 0:1]
    return 1.0 / jnp.maximum(deg, 1.0)


def _prep_body(x_ref, axp_ref, degp_ref, wxs_ref, wxn_ref, b_ref, p_ref):
    dinv = _dinv_block(degp_ref)
    aggx = (axp_ref[0, 0] + axp_ref[0, 1]) * dinv
    p_ref[0] = (jnp.dot(x_ref[0], wxs_ref[...],
                        preferred_element_type=jnp.float32)
                + jnp.dot(aggx, wxn_ref[...],
                          preferred_element_type=jnp.float32)
                + b_ref[...])


def _tc_prep(x, axp, degp, wxs, wxn, b):
    return pl.pallas_call(
        _prep_body,
        grid=(_T, _N // _R),
        in_specs=[
            pl.BlockSpec((1, _R, _D), lambda t, r: (t, r, 0)),
            pl.BlockSpec((1, 2, _R, _D), lambda t, r: (t, 0, r, 0)),
            pl.BlockSpec((2, _R, 1), lambda t, r: (0, r, 0)),
            pl.BlockSpec((_D, 2 * _D), lambda t, r: (0, 0)),
            pl.BlockSpec((_D, 2 * _D), lambda t, r: (0, 0)),
            pl.BlockSpec((1, 2 * _D), lambda t, r: (0, 0)),
        ],
        out_specs=pl.BlockSpec((1, _R, 2 * _D), lambda t, r: (t, r, 0)),
        out_shape=jax.ShapeDtypeStruct((_T, _N, 2 * _D), jnp.float32),
    )(x, axp, degp, wxs, wxn, b)


def _enc0_body(p_ref, h_ref):
    g = p_ref[...]
    u = jax.nn.sigmoid(g[:, :_D])
    cc = jnp.tanh(g[:, _D:])
    h_ref[...] = (1.0 - u) * cc


def _tc_enc0(p0):
    return pl.pallas_call(
        _enc0_body,
        grid=(_N // _R,),
        in_specs=[pl.BlockSpec((_R, 2 * _D), lambda r: (r, 0))],
        out_specs=pl.BlockSpec((_R, _D), lambda r: (r, 0)),
        out_shape=jax.ShapeDtypeStruct((_N, _D), jnp.float32),
    )(p0)


def _enc_body(p_ref, h_ref, ahp_ref, degp_ref, whs_ref, whn_ref, hn_ref):
    dinv = _dinv_block(degp_ref)
    aggh = (ahp_ref[0] + ahp_ref[1]) * dinv
    h = h_ref[...]
    g = (p_ref[...]
         + jnp.dot(h, whs_ref[...], preferred_element_type=jnp.float32)
         + jnp.dot(aggh, whn_ref[...], preferred_element_type=jnp.float32))
    u = jax.nn.sigmoid(g[:, :_D])
    cc = jnp.tanh(g[:, _D:])
    hn_ref[...] = u * h + (1.0 - u) * cc


def _tc_enc(p, h, ahp, degp, whs, whn):
    return pl.pallas_call(
        _enc_body,
        grid=(_N // _R,),
        in_specs=[
            pl.BlockSpec((_R, 2 * _D), lambda r: (r, 0)),
            pl.BlockSpec((_R, _D), lambda r: (r, 0)),
            pl.BlockSpec((2, _R, _D), lambda r: (0, r, 0)),
            pl.BlockSpec((2, _R, 1), lambda r: (0, r, 0)),
            pl.BlockSpec((_D, 2 * _D), lambda r: (0, 0)),
            pl.BlockSpec((_D, 2 * _D), lambda r: (0, 0)),
        ],
        out_specs=pl.BlockSpec((_R, _D), lambda r: (r, 0)),
        out_shape=jax.ShapeDtypeStruct((_N, _D), jnp.float32),
    )(p, h, ahp, degp, whs, whn)


def _dec_body(h_ref, ahp_ref, degp_ref, whs_ref, whn_ref, b_ref, hn_ref):
    dinv = _dinv_block(degp_ref)
    aggh = (ahp_ref[0] + ahp_ref[1]) * dinv
    h = h_ref[...]
    g = (jnp.dot(h, whs_ref[...], preferred_element_type=jnp.float32)
         + jnp.dot(aggh, whn_ref[...], preferred_element_type=jnp.float32)
         + b_ref[...])
    u = jax.nn.sigmoid(g[:, :_D])
    cc = jnp.tanh(g[:, _D:])
    hn_ref[...] = u * h + (1.0 - u) * cc


def _tc_dec(h, ahp, degp, whs, whn, b):
    return pl.pallas_call(
        _dec_body,
        grid=(_N // _R,),
        in_specs=[
            pl.BlockSpec((_R, _D), lambda r: (r, 0)),
            pl.BlockSpec((2, _R, _D), lambda r: (0, r, 0)),
            pl.BlockSpec((2, _R, 1), lambda r: (0, r, 0)),
            pl.BlockSpec((_D, 2 * _D), lambda r: (0, 0)),
            pl.BlockSpec((_D, 2 * _D), lambda r: (0, 0)),
            pl.BlockSpec((1, 2 * _D), lambda r: (0, 0)),
        ],
        out_specs=pl.BlockSpec((_R, _D), lambda r: (r, 0)),
        out_shape=jax.ShapeDtypeStruct((_N, _D), jnp.float32),
    )(h, ahp, degp, whs, whn, b)


def _out_body(hs_ref, ow_ref, ob_ref, y_ref):
    y_ref[0] = (jnp.dot(hs_ref[0], ow_ref[...],
                        preferred_element_type=jnp.float32) + ob_ref[...])


def _tc_out(hs, ow, ob):
    return pl.pallas_call(
        _out_body,
        grid=(_T, _N // _R),
        in_specs=[
            pl.BlockSpec((1, _R, _D), lambda t, r: (t, r, 0)),
            pl.BlockSpec((_D, _D), lambda t, r: (0, 0)),
            pl.BlockSpec((1, _D), lambda t, r: (0, 0)),
        ],
        out_specs=pl.BlockSpec((1, _R, _D), lambda t, r: (t, r, 0)),
        out_shape=jax.ShapeDtypeStruct((_T, _N, _D), jnp.float32),
    )(hs, ow, ob)


def _stack_uc(w):
    return jnp.concatenate([w[1], w[2]], axis=1)


def _bias_uc(bx, bh):
    return jnp.concatenate([bx[1] + bh[1], bx[2] + bh[2]])[None, :]


def kernel(edge_index, inputs, teacher_states, batch_cnt,
           enc_Wx_self, enc_Wx_neigh, enc_bx, enc_Wh_self, enc_Wh_neigh,
           enc_bh, dec_Wx_self, dec_Wx_neigh, dec_bx, dec_Wh_self,
           dec_Wh_neigh, dec_bh, out_W, out_b):
    src = edge_index[0].astype(jnp.int32)
    dst = edge_index[1].astype(jnp.int32)
    npad = _EPAD - _E
    ar = jnp.arange(npad, dtype=jnp.int32)
    srcp = jnp.concatenate([src, (ar * 131) % _N]).reshape(
        _NC * _NS, _NCHUNK, _CH)
    dstp = jnp.concatenate([dst, _N + (ar % (_NP - _N))]).reshape(
        _NC * _NS, _NCHUNK, _CH)
    zeros_np = jnp.zeros((_NP, _D), jnp.float32)

    degp = _sc_deg(dstp).reshape(_NC, _NP, 1)

    axp_list = _sc_aggx4([inputs[t] for t in range(_T)], srcp, dstp, zeros_np)
    axp = jnp.stack([a.reshape(_NC, _NP, _D) for a in axp_list])

    enc_wxs = _stack_uc(enc_Wx_self)
    enc_wxn = _stack_uc(enc_Wx_neigh)
    enc_whs = _stack_uc(enc_Wh_self)
    enc_whn = _stack_uc(enc_Wh_neigh)
    enc_b = _bias_uc(enc_bx, enc_bh)
    dec_whs = _stack_uc(dec_Wh_self)
    dec_whn = _stack_uc(dec_Wh_neigh)
    dec_b = _bias_uc(dec_bx, dec_bh)
    ow_t = out_W.T
    ob = out_b[None, :]

    p_all = _tc_prep(inputs, axp, degp, enc_wxs, enc_wxn, enc_b)

    h = _tc_enc0(p_all[0])
    for t in range(1, _T):
        ahp = _sc_agg(h, srcp, dstp, zeros_np).reshape(_NC, _NP, _D)
        h = _tc_enc(p_all[t], h, ahp, degp, enc_whs, enc_whn)

    hs = []
    for _ in range(_T):
        ahp = _sc_agg(h, srcp, dstp, zeros_np).reshape(_NC, _NP, _D)
        h = _tc_dec(h, ahp, degp, dec_whs, dec_whn, dec_b)
        hs.append(h)
    return _tc_out(jnp.stack(hs), ow_t, ob)
